# Initial kernel scaffold; baseline (speedup 1.0000x reference)
import jax
import jax.numpy as jnp
from jax.experimental import pallas as pl


def kernel(x, corner_idx, features, lod):
    raise NotImplementedError("write your pallas kernel here")


# speedup vs baseline: 1.6865x; 1.6865x over previous
import functools

import jax
import jax.numpy as jnp
from jax import lax
from jax.experimental import pallas as pl
from jax.experimental.pallas import tpu as pltpu
from jax.experimental.pallas import tpu_sc as plsc

_BASE_LOD = 9
_N = 262144
_V = 2000000
_D = 16
_L = 16

_P = 256
_CHUNK = 128
_RPB = _P * 8
_NCHUNK = _RPB // _CHUNK


@functools.lru_cache(maxsize=None)
def _make_kernel(nc, ns):
    nw = nc * ns
    n_per_w = _N // nw
    blocks = n_per_w // _P
    mesh = plsc.VectorSubcoreMesh(core_axis_name="c", subcore_axis_name="s")

    @functools.partial(
        pl.kernel,
        mesh=mesh,
        compiler_params=pltpu.CompilerParams(
            needs_layout_passes=False, use_tc_tiling_on_sc=False),
        out_type=jax.ShapeDtypeStruct((_N, _D), jnp.float32),
        scratch_types=[
            pltpu.VMEM((_NCHUNK, _CHUNK), jnp.int32),
            pltpu.VMEM((_RPB, _D), jnp.float32),
            pltpu.VMEM((_P * 3,), jnp.float32),
            pltpu.VMEM((_RPB,), jnp.float32),
            pltpu.VMEM((_P, _D), jnp.float32),
            pltpu.VMEM((_L,), jnp.float32),
            pltpu.SemaphoreType.DMA,
        ],
    )
    def spc_kernel(x_hbm, cidx_hbm, feat_hbm, res_hbm, out_hbm,
                   idx_v, rows_v, x_v, coeff_v, out_v, res_v, sem):
        wid = lax.axis_index("s") * nc + lax.axis_index("c")
        pltpu.sync_copy(res_hbm, res_v)
        res = res_v[...]
        lane = lax.iota(jnp.int32, _L)

        def block_body(b, carry):
            pbase = pl.multiple_of(wid * n_per_w + b * _P, _P)
            cbase = pl.multiple_of(pbase * 8 // _CHUNK, _P * 8 // _CHUNK)
            pltpu.sync_copy(x_hbm.at[pl.ds(pbase * 3, _P * 3)], x_v)
            pltpu.sync_copy(cidx_hbm.at[pl.ds(cbase, _NCHUNK)], idx_v)
            copies = [
                pltpu.async_copy(
                    feat_hbm.at[idx_v.at[c]],
                    rows_v.at[pl.ds(c * _CHUNK, _CHUNK)],
                    sem,
                )
                for c in range(_NCHUNK)
            ]

            def grp_body(g, carry2):
                xbase = (g * _L + lane) * 3
                f0 = plsc.load_gather(x_v, [xbase]) * res
                f1 = plsc.load_gather(x_v, [xbase + 1]) * res
                f2 = plsc.load_gather(x_v, [xbase + 2]) * res
                f0 = f0 - f0.astype(jnp.int32).astype(jnp.float32)
                f1 = f1 - f1.astype(jnp.int32).astype(jnp.float32)
                f2 = f2 - f2.astype(jnp.int32).astype(jnp.float32)
                a0 = 1.0 - f0
                a1 = 1.0 - f1
                a2 = 1.0 - f2
                p00 = a0 * a1
                p01 = a0 * f1
                p10 = f0 * a1
                p11 = f0 * f1
                cs = (p00 * a2, p00 * f2, p01 * a2, p01 * f2,
                      p10 * a2, p10 * f2, p11 * a2, p11 * f2)
                sbase = lane * 8 + g * (_L * 8)
                for j in range(8):
                    plsc.store_scatter(coeff_v, [sbase + j], cs[j])
                return carry2

            lax.fori_loop(0, _P // _L, grp_body, 0)
            for cp in copies:
                cp.wait()

            def pair_body(m, carry2):
                r = m * 16
                cv = coeff_v[pl.ds(r, _L)]
                acc0 = cv[0] * rows_v[r]
                acc1 = cv[8] * rows_v[r + 8]
                for j in range(1, 8):
                    acc0 = acc0 + cv[j] * rows_v[r + j]
                    acc1 = acc1 + cv[8 + j] * rows_v[r + 8 + j]
                out_v[2 * m] = acc0
                out_v[2 * m + 1] = acc1
                return carry2

            lax.fori_loop(0, _P // 2, pair_body, 0)
            pltpu.sync_copy(out_v, out_hbm.at[pl.ds(pbase, _P)])
            return carry

        lax.fori_loop(0, blocks, block_body, 0)

    return spc_kernel


def kernel(x, corner_idx, features, lod):
    res = (jnp.asarray(2, jnp.int32) ** (lod + _BASE_LOD)).astype(jnp.float32)
    res_vec = jnp.full((_L,), 1.0, jnp.float32) * res
    x_flat = x.reshape(_N * 3)
    cidx2d = corner_idx.reshape(_N * 8 // _CHUNK, _CHUNK)
    info = plsc.get_sparse_core_info()
    k = _make_kernel(info.num_cores, info.num_subcores)
    return k(x_flat, cidx2d, features, res_vec)
